# Initial kernel scaffold; baseline (speedup 1.0000x reference)
import jax
import jax.numpy as jnp
from jax.experimental import pallas as pl


def kernel(x, W):
    raise NotImplementedError("write your pallas kernel here")


# speedup vs baseline: 3.3687x; 3.3687x over previous
import functools

import jax
import jax.numpy as jnp
import numpy as np
from jax import lax
from jax.experimental import pallas as pl
from jax.experimental.pallas import tpu as pltpu
from jax.experimental.pallas import tpu_sc as plsc

_VOCAB = 100000
_D = 64
_SEQ = 200
_BATCH = 4096
_TOKENS = _BATCH * _SEQ
_NC, _NS, _L = 2, 16, 16
_NW = _NC * _NS
_TOK_PER_W = _TOKENS // _NW
_CT = 400
_NCHUNK = _TOK_PER_W // _CT


def _pe_table() -> np.ndarray:
    position = np.arange(_SEQ, dtype=np.float32)[:, None]
    div = np.exp(np.arange(0, _D, 2, dtype=np.float32) * -(np.log(10000.0) / _D))
    pe = np.zeros((_SEQ, _D), np.float32)
    pe[:, 0::2] = np.sin(position * div)
    pe[:, 1::2] = np.cos(position * div)
    return np.tile(pe, (_CT // _SEQ, 1))


_PE = _pe_table()


@functools.partial(
    pl.kernel,
    mesh=plsc.VectorSubcoreMesh(core_axis_name="c", subcore_axis_name="s"),
    out_type=jax.ShapeDtypeStruct((_TOKENS, _D), jnp.float32),
    scratch_types=[
        pltpu.VMEM((_CT,), jnp.int32),
        pltpu.VMEM((_CT, _D), jnp.float32),
        pltpu.VMEM((_CT, _D), jnp.float32),
        pltpu.SemaphoreType.DMA,
    ],
    compiler_params=pltpu.CompilerParams(use_tc_tiling_on_sc=False),
)
def _emb_kernel(x_hbm, w_hbm, pe_hbm, out_hbm, idx_v, rows_v, pe_v, sem):
    wid = lax.axis_index("s") * _NC + lax.axis_index("c")
    base = wid * _TOK_PER_W
    pltpu.sync_copy(pe_hbm, pe_v)

    def chunk_body(c, carry):
        tok0 = base + c * _CT
        pltpu.sync_copy(x_hbm.at[pl.ds(tok0, _CT)], idx_v)
        pltpu.async_copy(w_hbm.at[idx_v], rows_v, sem).wait()

        def add_body(i, acc):
            for k in range(_D // _L):
                sl = pl.ds(k * _L, _L)
                rows_v[i, sl] += pe_v[i, sl]
            return acc

        lax.fori_loop(0, _CT, add_body, 0)
        pltpu.sync_copy(rows_v, out_hbm.at[pl.ds(tok0, _CT)])
        return carry

    lax.fori_loop(0, _NCHUNK, chunk_body, 0)


def kernel(x, W):
    xf = x.reshape(_TOKENS).astype(jnp.int32)
    out = _emb_kernel(xf, W, jnp.asarray(_PE))
    return out.reshape(_BATCH, _SEQ, _D)


# speedup vs baseline: 4.2458x; 1.2604x over previous
import functools

import jax
import jax.numpy as jnp
import numpy as np
from jax import lax
from jax.experimental import pallas as pl
from jax.experimental.pallas import tpu as pltpu
from jax.experimental.pallas import tpu_sc as plsc

_VOCAB = 100000
_D = 64
_SEQ = 200
_BATCH = 4096
_TOKENS = _BATCH * _SEQ
_NC, _NS, _L = 2, 16, 16
_NW = _NC * _NS
_TOK_PER_W = _TOKENS // _NW
_CT = 200
_NCHUNK = _TOK_PER_W // _CT
_NBUF = 4
_LEAD = 2
_NG = _NCHUNK // _NBUF


def _pe_table() -> np.ndarray:
    position = np.arange(_SEQ, dtype=np.float32)[:, None]
    div = np.exp(np.arange(0, _D, 2, dtype=np.float32) * -(np.log(10000.0) / _D))
    pe = np.zeros((_SEQ, _D), np.float32)
    pe[:, 0::2] = np.sin(position * div)
    pe[:, 1::2] = np.cos(position * div)
    return np.tile(pe, (_CT // _SEQ, 1))


_PE = _pe_table()


@functools.partial(
    pl.kernel,
    mesh=plsc.VectorSubcoreMesh(core_axis_name="c", subcore_axis_name="s"),
    out_type=jax.ShapeDtypeStruct((_TOKENS, _D), jnp.float32),
    scratch_types=[
        pltpu.VMEM((_NCHUNK, _CT), jnp.int32),
        pltpu.VMEM((_NBUF, _CT, _D), jnp.float32),
        pltpu.VMEM((_CT, _D), jnp.float32),
        pltpu.SemaphoreType.DMA((_NBUF,)),
        pltpu.SemaphoreType.DMA((_NBUF,)),
    ],
    compiler_params=pltpu.CompilerParams(use_tc_tiling_on_sc=False),
)
def _emb_kernel(x_hbm, w_hbm, pe_hbm, out_hbm, idx_v, rows_v, pe_v, sem_g, sem_o):
    wid = lax.axis_index("s") * _NC + lax.axis_index("c")
    base = wid * _TOK_PER_W

    pltpu.sync_copy(x_hbm.at[wid], idx_v)
    pltpu.sync_copy(pe_hbm, pe_v)

    def start_gather(c, slot):
        pltpu.async_copy(w_hbm.at[idx_v.at[c]], rows_v.at[slot], sem_g.at[slot])

    def wait_gather(c, slot):
        pltpu.make_async_copy(
            w_hbm.at[idx_v.at[c]], rows_v.at[slot], sem_g.at[slot]).wait()

    def start_out(c, slot):
        pltpu.async_copy(
            rows_v.at[slot], out_hbm.at[pl.ds(base + c * _CT, _CT)],
            sem_o.at[slot])

    def wait_out(c, slot):
        pltpu.make_async_copy(
            rows_v.at[slot], out_hbm.at[pl.ds(base + c * _CT, _CT)],
            sem_o.at[slot]).wait()

    def add_pe(slot):
        def add_body(i, acc):
            for k in range(_D // _L):
                sl = pl.ds(k * _L, _L)
                rows_v[slot, i, sl] += pe_v[i, sl]
            return acc
        lax.fori_loop(0, _CT, add_body, 0)

    def emit(c, b, do_wait_out, do_prefetch):
        pslot = (b + _LEAD) % _NBUF
        if do_wait_out:
            wait_out(c - _LEAD, pslot)
        if do_prefetch:
            start_gather(c + _LEAD, pslot)
        wait_gather(c, b)
        add_pe(b)
        start_out(c, b)

    for c0 in range(_LEAD):
        start_gather(c0, c0 % _NBUF)

    for b in range(_NBUF):
        emit(b, b, do_wait_out=(b >= _LEAD), do_prefetch=True)

    def group_body(g, acc):
        c0 = g * _NBUF
        for b in range(_NBUF):
            emit(c0 + b, b, do_wait_out=True, do_prefetch=True)
        return acc

    lax.fori_loop(1, _NG - 1, group_body, 0)

    cl = (_NG - 1) * _NBUF
    for b in range(_NBUF):
        pref = cl + b + _LEAD < _NCHUNK
        emit(cl + b, b, do_wait_out=pref, do_prefetch=pref)

    for b in range(_NBUF):
        wait_out(cl + b, b)


def kernel(x, W):
    xr = x.reshape(_NW, _NCHUNK, _CT).astype(jnp.int32)
    out = _emb_kernel(xr, W, jnp.asarray(_PE))
    return out.reshape(_BATCH, _SEQ, _D)


# speedup vs baseline: 7.4890x; 1.7639x over previous
import functools

import jax
import jax.numpy as jnp
import numpy as np
from jax import lax
from jax.experimental import pallas as pl
from jax.experimental.pallas import tpu as pltpu
from jax.experimental.pallas import tpu_sc as plsc

_VOCAB = 100000
_D = 64
_SEQ = 200
_BATCH = 4096
_TOKENS = _BATCH * _SEQ
_NC, _NS, _L = 2, 16, 16
_NW = _NC * _NS
_TOK_PER_W = _TOKENS // _NW
_CT = 200
_NCHUNK = _TOK_PER_W // _CT
_NBUF = 4
_LEAD = 2
_NG = _NCHUNK // _NBUF


def _pe_table() -> np.ndarray:
    position = np.arange(_SEQ, dtype=np.float32)[:, None]
    div = np.exp(np.arange(0, _D, 2, dtype=np.float32) * -(np.log(10000.0) / _D))
    pe = np.zeros((_SEQ, _D), np.float32)
    pe[:, 0::2] = np.sin(position * div)
    pe[:, 1::2] = np.cos(position * div)
    return np.tile(pe, (_CT // _SEQ, 1))


_PE = _pe_table()


@functools.partial(
    pl.kernel,
    mesh=plsc.VectorSubcoreMesh(core_axis_name="c", subcore_axis_name="s"),
    out_type=jax.ShapeDtypeStruct((_TOKENS, 128), jnp.float32),
    scratch_types=[
        pltpu.VMEM((_NCHUNK, _CT), jnp.int32),
        pltpu.VMEM((_NBUF, _CT, _D), jnp.float32),
        pltpu.VMEM((_CT, _D), jnp.float32),
        pltpu.SemaphoreType.DMA((_NBUF,)),
        pltpu.SemaphoreType.DMA((_NBUF,)),
    ],
    compiler_params=pltpu.CompilerParams(use_tc_tiling_on_sc=False),
)
def _emb_kernel(x_hbm, w_hbm, pe_hbm, out_hbm, idx_v, rows_v, pe_v, sem_g, sem_o):
    wid = lax.axis_index("s") * _NC + lax.axis_index("c")
    base = wid * _TOK_PER_W

    pltpu.sync_copy(x_hbm.at[wid], idx_v)
    pltpu.sync_copy(pe_hbm, pe_v)

    def start_gather(c, slot):
        pltpu.async_copy(w_hbm.at[idx_v.at[c]], rows_v.at[slot], sem_g.at[slot])

    def wait_gather(c, slot):
        pltpu.make_async_copy(
            w_hbm.at[idx_v.at[c]], rows_v.at[slot], sem_g.at[slot]).wait()

    def start_out(c, slot):
        pltpu.async_copy(
            rows_v.at[slot],
            out_hbm.at[pl.ds(base + c * _CT, _CT), pl.ds(0, _D)],
            sem_o.at[slot])

    def wait_out(c, slot):
        pltpu.make_async_copy(
            rows_v.at[slot],
            out_hbm.at[pl.ds(base + c * _CT, _CT), pl.ds(0, _D)],
            sem_o.at[slot]).wait()

    def add_pe(slot):
        def add_body(i, acc):
            for k in range(_D // _L):
                sl = pl.ds(k * _L, _L)
                rows_v[slot, i, sl] += pe_v[i, sl]
            return acc
        lax.fori_loop(0, _CT, add_body, 0)

    def emit(c, b, do_wait_out, do_prefetch):
        pslot = (b + _LEAD) % _NBUF
        if do_wait_out:
            wait_out(c - _LEAD, pslot)
        if do_prefetch:
            start_gather(c + _LEAD, pslot)
        wait_gather(c, b)
        add_pe(b)
        start_out(c, b)

    for c0 in range(_LEAD):
        start_gather(c0, c0 % _NBUF)

    for b in range(_NBUF):
        emit(b, b, do_wait_out=(b >= _LEAD), do_prefetch=True)

    def group_body(g, acc):
        c0 = g * _NBUF
        for b in range(_NBUF):
            emit(c0 + b, b, do_wait_out=True, do_prefetch=True)
        return acc

    lax.fori_loop(1, _NG - 1, group_body, 0)

    cl = (_NG - 1) * _NBUF
    for b in range(_NBUF):
        pref = cl + b + _LEAD < _NCHUNK
        emit(cl + b, b, do_wait_out=pref, do_prefetch=pref)

    for b in range(_NBUF):
        wait_out(cl + b, b)


def kernel(x, W):
    xr = x.reshape(_NW, _NCHUNK, _CT).astype(jnp.int32)
    out = _emb_kernel(xr, W, jnp.asarray(_PE))
    return out.reshape(_BATCH, _SEQ, 128)[:, :, :_D]
